# baseline (speedup 1.0000x reference)
import jax
import jax.numpy as jnp
from jax import lax
from jax.experimental import pallas as pl

_BR = 256


def _tc_body(x_ref, o_ref):
    v = x_ref[...]
    br = v.shape[1]
    g = jnp.concatenate(
        [v[:, br - 8 * (j + 1):br - 8 * j, :] for j in range(br // 8)], axis=1
    )
    i = lax.broadcasted_iota(jnp.int32, g.shape, 1)
    for k in (1, 2, 4):
        g = jnp.where((i & k) == 0, jnp.roll(g, -k, axis=1), jnp.roll(g, k, axis=1))
    o_ref[...] = g


def tc_reverse(x):
    B, N, D = x.shape
    nb = N // _BR
    return pl.pallas_call(
        _tc_body,
        grid=(B, nb),
        in_specs=[pl.BlockSpec((1, _BR, D), lambda b, i: (b, nb - 1 - i, 0))],
        out_specs=pl.BlockSpec((1, _BR, D), lambda b, i: (b, i, 0)),
        out_shape=jax.ShapeDtypeStruct(x.shape, x.dtype),
    )(x)


def kernel(x):
    return tc_reverse(x)
